# baseline (speedup 1.0000x reference)
import functools

import jax
import jax.numpy as jnp
from jax import lax
from jax.experimental import pallas as pl
from jax.experimental.pallas import tpu as pltpu
from jax.experimental.pallas import tpu_sc as plsc

NUM_SYSTEMS = 512
N_ATOMS = 50000
N_PAIRS = 1600000

NC = 2
NS = 16
NW = NC * NS
L = 16

BLK = 128
NBLK = N_PAIRS // BLK
BLK_PER_W = NBLK // NW
EXTRA_W = NBLK - BLK_PER_W * NW
CHUNK_BLK = 15
CHUNK = CHUNK_BLK * BLK
NCHUNKS = BLK_PER_W // CHUNK_BLK
VECS = CHUNK // L
UNROLL = 6

COULOMB_SCALE = 138.96

RC0 = 0.9997018290052982
RC1 = -0.48917531316256574
RC2 = 0.2748334539844737


@functools.partial(
    pl.kernel,
    out_type=jax.ShapeDtypeStruct((NW, NUM_SYSTEMS), jnp.float32),
    mesh=plsc.VectorSubcoreMesh(core_axis_name="c", subcore_axis_name="s"),
    compiler_params=pltpu.CompilerParams(needs_layout_passes=False),
    scratch_types=[
        pltpu.VMEM((N_ATOMS,), jnp.float32),
        pltpu.VMEM((N_ATOMS,), jnp.int32),
        pltpu.VMEM((NUM_SYSTEMS,), jnp.float32),
        pltpu.VMEM((NUM_SYSTEMS,), jnp.float32),
        pltpu.VMEM((2, 2, CHUNK), jnp.int32),
        pltpu.VMEM((2 * CHUNK,), jnp.float32),
        pltpu.VMEM((2, BLK), jnp.int32),
        pltpu.VMEM((BLK,), jnp.float32),
        pltpu.SemaphoreType.DMA,
    ],
)
def _sc_coulomb(q_hbm, sys_hbm, pairs_hbm, d_hbm, out_hbm,
                q_v, sys_v, acc, res, pb_v, d_v, tp_v, td_v, sem):
    cid = lax.axis_index("c")
    sid = lax.axis_index("s")
    wid = sid * NC + cid
    base_blk = wid * BLK_PER_W + jnp.minimum(wid, EXTRA_W)
    base_w = base_blk * BLK

    def issue_chunk(ci):
        cb = pl.multiple_of(base_w + ci * CHUNK, BLK)
        par = lax.rem(ci, 2)
        pltpu.async_copy(pairs_hbm.at[:, pl.ds(cb, CHUNK)], pb_v.at[par], sem)
        pltpu.async_copy(d_hbm.at[pl.ds(cb, CHUNK)],
                         d_v.at[pl.ds(par * CHUNK, CHUNK)], sem)

    def wait_chunk(ci):
        cb = pl.multiple_of(base_w + ci * CHUNK, BLK)
        par = lax.rem(ci, 2)
        pltpu.make_async_copy(pairs_hbm.at[:, pl.ds(cb, CHUNK)],
                              pb_v.at[par], sem).wait()
        pltpu.make_async_copy(d_hbm.at[pl.ds(cb, CHUNK)],
                              d_v.at[pl.ds(par * CHUNK, CHUNK)], sem).wait()

    q_copy = pltpu.async_copy(q_hbm, q_v, sem)
    sys_copy = pltpu.async_copy(sys_hbm, sys_v, sem)
    issue_chunk(0)

    zero = jnp.zeros((L,), jnp.float32)

    def zero_body(i, _):
        acc[pl.ds(i * L, L)] = zero
        return 0

    lax.fori_loop(0, NUM_SYSTEMS // L, zero_body, 0)

    q_copy.wait()
    sys_copy.wait()

    def process(iis, jjs, dds):
        rcs = [1.0 / dd for dd in dds]
        qis = [plsc.load_gather(q_v, [ii]) for ii in iis]
        qjs = [plsc.load_gather(q_v, [jj]) for jj in jjs]
        segs = [plsc.load_gather(sys_v, [ii]) for ii in iis]
        us = [dd * dd for dd in dds]
        aa = [240.0 - 192.0 * dd for dd in dds]
        bb = [dd * a for dd, a in zip(dds, aa)]
        cc = [b - 80.0 for b in bb]
        dus = [dd * u for dd, u in zip(dds, us)]
        pps = [du * c for du, c in zip(dus, cc)]
        rr = [RC0 + u * (RC1 + RC2 * u) for u in us]
        mms = [r - rc for r, rc in zip(rr, rcs)]
        phis = [jnp.maximum(1.0 + p, 0.0) for p in pps]
        chis = [rc + m * phi for rc, m, phi in zip(rcs, mms, phis)]
        es = [qi * qj * chi for qi, qj, chi in zip(qis, qjs, chis)]
        msks = [ii < jj for ii, jj in zip(iis, jjs)]
        for seg, e, msk in zip(segs, es, msks):
            plsc.addupdate_scatter(acc, [seg], e, mask=msk)

    def chunk_body(ci, _):
        wait_chunk(ci)

        @pl.when(ci + 1 < NCHUNKS)
        def _():
            issue_chunk(ci + 1)

        par = lax.rem(ci, 2)
        doff = par * CHUNK

        @plsc.parallel_loop(0, VECS // UNROLL)
        def vec_body(vi):
            o = vi * (L * UNROLL)
            ks = range(UNROLL)
            process([pb_v[par, 0, pl.ds(o + k * L, L)] for k in ks],
                    [pb_v[par, 1, pl.ds(o + k * L, L)] for k in ks],
                    [d_v[pl.ds(doff + o + k * L, L)] for k in ks])

        return 0

    lax.fori_loop(0, NCHUNKS, chunk_body, 0)

    @pl.when(wid < EXTRA_W)
    def _():
        tb = pl.multiple_of((base_blk + BLK_PER_W) * BLK, BLK)
        pltpu.sync_copy(pairs_hbm.at[:, pl.ds(tb, BLK)], tp_v)
        pltpu.sync_copy(d_hbm.at[pl.ds(tb, BLK)], td_v)

        def tail_body(vi, _):
            o = vi * L
            process([tp_v[0, pl.ds(o, L)]],
                    [tp_v[1, pl.ds(o, L)]],
                    [td_v[pl.ds(o, L)]])
            return 0

        lax.fori_loop(0, BLK // L, tail_body, 0)

    def fold_body(ci, _):
        o = ci * L
        res[pl.ds(o, L)] = acc[pl.ds(o, L)] * COULOMB_SCALE
        return 0

    lax.fori_loop(0, NUM_SYSTEMS // L, fold_body, 0)
    pltpu.sync_copy(res, out_hbm.at[wid])


def _combine_body(p_ref, o_ref):
    o_ref[...] = jnp.sum(p_ref[...], axis=0)


def kernel(per_atom_charge, d_ij, pair_indices, atomic_subsystem_indices):
    partials = _sc_coulomb(per_atom_charge, atomic_subsystem_indices,
                           pair_indices, d_ij)
    total = pl.pallas_call(
        _combine_body,
        out_shape=jax.ShapeDtypeStruct((NUM_SYSTEMS,), jnp.float32),
    )(partials)
    return total[:, None]


# speedup vs baseline: 1.2103x; 1.2103x over previous
import functools

import jax
import jax.numpy as jnp
from jax import lax
from jax.experimental import pallas as pl
from jax.experimental.pallas import tpu as pltpu
from jax.experimental.pallas import tpu_sc as plsc

NUM_SYSTEMS = 512
N_ATOMS = 50000
N_PAIRS = 1600000

NC = 2
NS = 16
NW = NC * NS
L = 16

BLK = 128
NBLK = N_PAIRS // BLK
BLK_PER_W = NBLK // NW
EXTRA_W = NBLK - BLK_PER_W * NW
CHUNK_BLK = 39
CHUNK = CHUNK_BLK * BLK
NCHUNKS = BLK_PER_W // CHUNK_BLK
VECS = CHUNK // L
UNROLL = 6

COULOMB_SCALE = 138.96

RC0 = 0.9997018290052982
RC1 = -0.48917531316256574
RC2 = 0.2748334539844737


@functools.partial(
    pl.kernel,
    out_type=jax.ShapeDtypeStruct((NW, NUM_SYSTEMS), jnp.float32),
    mesh=plsc.VectorSubcoreMesh(core_axis_name="c", subcore_axis_name="s"),
    compiler_params=pltpu.CompilerParams(needs_layout_passes=False),
    scratch_types=[
        pltpu.VMEM((N_ATOMS,), jnp.int32),
        pltpu.VMEM((NUM_SYSTEMS,), jnp.float32),
        pltpu.VMEM((NUM_SYSTEMS,), jnp.float32),
        pltpu.VMEM((2, 2, CHUNK), jnp.int32),
        pltpu.VMEM((2 * CHUNK,), jnp.float32),
        pltpu.VMEM((2, BLK), jnp.int32),
        pltpu.VMEM((BLK,), jnp.float32),
        pltpu.SemaphoreType.DMA,
    ],
)
def _sc_coulomb(tab_hbm, pairs_hbm, d_hbm, out_hbm,
                tab_v, acc, res, pb_v, d_v, tp_v, td_v, sem):
    cid = lax.axis_index("c")
    sid = lax.axis_index("s")
    wid = sid * NC + cid
    base_blk = wid * BLK_PER_W + jnp.minimum(wid, EXTRA_W)
    base_w = base_blk * BLK

    def issue_chunk(ci):
        cb = pl.multiple_of(base_w + ci * CHUNK, BLK)
        par = lax.rem(ci, 2)
        pltpu.async_copy(pairs_hbm.at[:, pl.ds(cb, CHUNK)], pb_v.at[par], sem)
        pltpu.async_copy(d_hbm.at[pl.ds(cb, CHUNK)],
                         d_v.at[pl.ds(par * CHUNK, CHUNK)], sem)

    def wait_chunk(ci):
        cb = pl.multiple_of(base_w + ci * CHUNK, BLK)
        par = lax.rem(ci, 2)
        pltpu.make_async_copy(pairs_hbm.at[:, pl.ds(cb, CHUNK)],
                              pb_v.at[par], sem).wait()
        pltpu.make_async_copy(d_hbm.at[pl.ds(cb, CHUNK)],
                              d_v.at[pl.ds(par * CHUNK, CHUNK)], sem).wait()

    tab_copy = pltpu.async_copy(tab_hbm, tab_v, sem)
    issue_chunk(0)

    zero = jnp.zeros((L,), jnp.float32)

    def zero_body(i, _):
        acc[pl.ds(i * L, L)] = zero
        return 0

    lax.fori_loop(0, NUM_SYSTEMS // L, zero_body, 0)

    tab_copy.wait()

    def process(iis, jjs, dds):
        rcs = [1.0 / dd for dd in dds]
        wis = [plsc.load_gather(tab_v, [ii]) for ii in iis]
        wjs = [plsc.load_gather(tab_v, [jj]) for jj in jjs]
        segs = [w & 0x1FF for w in wis]
        sjs = [w & 0x1FF for w in wjs]
        qis = [plsc.bitcast(w ^ s, jnp.float32) for w, s in zip(wis, segs)]
        qjs = [plsc.bitcast(w ^ s, jnp.float32) for w, s in zip(wjs, sjs)]
        us = [dd * dd for dd in dds]
        aa = [240.0 - 192.0 * dd for dd in dds]
        bb = [dd * a for dd, a in zip(dds, aa)]
        cc = [b - 80.0 for b in bb]
        dus = [dd * u for dd, u in zip(dds, us)]
        pps = [du * c for du, c in zip(dus, cc)]
        rr = [RC0 + u * (RC1 + RC2 * u) for u in us]
        mms = [r - rc for r, rc in zip(rr, rcs)]
        phis = [jnp.maximum(1.0 + p, 0.0) for p in pps]
        chis = [rc + m * phi for rc, m, phi in zip(rcs, mms, phis)]
        es = [qi * qj * chi for qi, qj, chi in zip(qis, qjs, chis)]
        msks = [ii < jj for ii, jj in zip(iis, jjs)]
        for seg, e, msk in zip(segs, es, msks):
            plsc.addupdate_scatter(acc, [seg], e, mask=msk)

    def chunk_body(ci, _):
        wait_chunk(ci)

        @pl.when(ci + 1 < NCHUNKS)
        def _():
            issue_chunk(ci + 1)

        par = lax.rem(ci, 2)
        doff = par * CHUNK

        @plsc.parallel_loop(0, VECS // UNROLL)
        def vec_body(vi):
            o = vi * (L * UNROLL)
            ks = range(UNROLL)
            process([pb_v[par, 0, pl.ds(o + k * L, L)] for k in ks],
                    [pb_v[par, 1, pl.ds(o + k * L, L)] for k in ks],
                    [d_v[pl.ds(doff + o + k * L, L)] for k in ks])

        return 0

    lax.fori_loop(0, NCHUNKS, chunk_body, 0)

    @pl.when(wid < EXTRA_W)
    def _():
        tb = pl.multiple_of((base_blk + BLK_PER_W) * BLK, BLK)
        pltpu.sync_copy(pairs_hbm.at[:, pl.ds(tb, BLK)], tp_v)
        pltpu.sync_copy(d_hbm.at[pl.ds(tb, BLK)], td_v)

        def tail_body(vi, _):
            o = vi * L
            process([tp_v[0, pl.ds(o, L)]],
                    [tp_v[1, pl.ds(o, L)]],
                    [td_v[pl.ds(o, L)]])
            return 0

        lax.fori_loop(0, BLK // L, tail_body, 0)

    def fold_body(ci, _):
        o = ci * L
        res[pl.ds(o, L)] = acc[pl.ds(o, L)] * COULOMB_SCALE
        return 0

    lax.fori_loop(0, NUM_SYSTEMS // L, fold_body, 0)
    pltpu.sync_copy(res, out_hbm.at[wid])


def _combine_body(p_ref, o_ref):
    o_ref[...] = jnp.sum(p_ref[...], axis=0)


def kernel(per_atom_charge, d_ij, pair_indices, atomic_subsystem_indices):
    packed = (jax.lax.bitcast_convert_type(per_atom_charge, jnp.int32)
              & jnp.int32(~0x1FF)) | atomic_subsystem_indices
    partials = _sc_coulomb(packed, pair_indices, d_ij)
    total = pl.pallas_call(
        _combine_body,
        out_shape=jax.ShapeDtypeStruct((NUM_SYSTEMS,), jnp.float32),
    )(partials)
    return total[:, None]
